# Initial kernel scaffold; baseline (speedup 1.0000x reference)
import jax
import jax.numpy as jnp
from jax.experimental import pallas as pl


def kernel(x, edge_index, embed, W1, b1, W2, b2, W3, b3, Wmu, bmu, Wls, bls):
    raise NotImplementedError("write your pallas kernel here")


# speedup vs baseline: 14.5472x; 14.5472x over previous
import functools

import jax
import jax.numpy as jnp
from jax import lax
from jax.experimental import pallas as pl
from jax.experimental.pallas import tpu as pltpu
from jax.experimental.pallas import tpu_sc as plsc

N = 50000
E = 800000
NUM_TYPES = 28
H = 64
OUT = 32
QW = 16

BN = 512
NP = 50176
NBLK = NP // BN
NT = NP
RT = NT // 16
EP = 802816
EPT = EP // 16
CE = 128
CR = 64
IB = 8
NIB = EPT // (CE * IB)
HCH = 196


_MESH = plsc.VectorSubcoreMesh(core_axis_name="c", subcore_axis_name="s")


NRC = (RT + CR - 1) // CR


def _build_zidx(zidx, base, n):
    iot = lax.broadcasted_iota(jnp.int32, (16,), 0)

    def st(j, carry):
        zidx[pl.ds(j * 16, 16)] = iot + (base + j * 16)
        return carry

    lax.fori_loop(0, n // 16, st, 0)


@functools.partial(
    pl.kernel,
    mesh=_MESH,
    out_type=jax.ShapeDtypeStruct((2, NP, QW), jnp.float32),
    scratch_types=[
        pltpu.VMEM_SHARED((NT, QW), jnp.float32),
        pltpu.VMEM((HCH, 128), jnp.int32),
        pltpu.VMEM((128, QW), jnp.float32),
        pltpu.VMEM((CR, QW), jnp.float32),
        pltpu.VMEM((CR,), jnp.int32),
        pltpu.SemaphoreType.DMA,
    ],
)
def _deg_sc(dst_hbm, out_hbm, acc, didx, ob, zbuf, zidx, sem):
    c = lax.axis_index("c")
    s = lax.axis_index("s")
    pltpu.sync_copy(dst_hbm.at[s, c], didx)

    ones = jnp.ones((16,), jnp.float32)
    zero = jnp.zeros((16,), jnp.float32)

    def fill(i, carry):
        ob[i, pl.ds(0, 16)] = ones
        return carry

    lax.fori_loop(0, 128, fill, 0)

    def zb(i, carry):
        zbuf[i, pl.ds(0, 16)] = zero
        return carry

    lax.fori_loop(0, CR, zb, 0)

    def zacc(i, carry):
        base = s * RT + i * CR
        _build_zidx(zidx, base, CR)
        pltpu.sync_copy(zbuf, acc.at[zidx])
        return carry

    lax.fori_loop(0, NRC, zacc, 0)
    plsc.subcore_barrier()

    def body(i, carry):
        pltpu.sync_copy(ob, acc.at[didx.at[i]], add=True)
        return carry

    lax.fori_loop(0, HCH, body, 0)
    plsc.subcore_barrier()

    def wout(i, carry):
        base = s * RT + i * CR
        _build_zidx(zidx, base, CR)
        pltpu.async_copy(acc.at[zidx], zbuf, sem).wait()
        pltpu.sync_copy(zbuf, out_hbm.at[c, pl.ds(base, CR)])
        return carry

    lax.fori_loop(0, NRC, wout, 0)


def _make_scatter(nq):
    pq = nq // 2

    @functools.partial(
        pl.kernel,
        mesh=_MESH,
        out_type=jax.ShapeDtypeStruct((nq, 16, NRC, CR, QW), jnp.float32),
        scratch_types=[
            pltpu.VMEM_SHARED((NT, QW), jnp.float32),
            pltpu.VMEM_SHARED((NT, QW), jnp.float32),
            pltpu.VMEM((IB, CE), jnp.int32),
            pltpu.VMEM((IB, CE), jnp.int32),
            pltpu.VMEM((CE, QW), jnp.float32),
            pltpu.VMEM((CR, QW), jnp.float32),
            pltpu.VMEM((CR,), jnp.int32),
            pltpu.SemaphoreType.DMA,
        ],
    )
    def scat(g_hbm, src_hbm, dst_hbm, out_hbm,
             stab, acc, sblk, dblk, rows, srow, zidx, sem0):
        c = lax.axis_index("c")
        s = lax.axis_index("s")

        for p in range(pq):
            q = c * pq + p
            zero = jnp.zeros((16,), jnp.float32)

            def zr(i, carry):
                srow[i, pl.ds(0, 16)] = zero
                return carry

            lax.fori_loop(0, CR, zr, 0)

            def zacc(i, carry):
                base = s * RT + i * CR
                _build_zidx(zidx, base, CR)
                pltpu.sync_copy(srow, acc.at[zidx])
                return carry

            lax.fori_loop(0, NRC, zacc, 0)

            def stage(i, carry):
                base = s * RT + i * CR
                _build_zidx(zidx, base, CR)
                pltpu.sync_copy(g_hbm.at[q, s, i], srow)
                pltpu.sync_copy(srow, stab.at[zidx])
                return carry

            lax.fori_loop(0, NRC, stage, 0)
            plsc.subcore_barrier()

            def blk(b, carry):
                pltpu.sync_copy(src_hbm.at[s, b], sblk)
                pltpu.sync_copy(dst_hbm.at[s, b], dblk)
                for k in range(IB):
                    pltpu.async_copy(
                        stab.at[sblk.at[k]], rows, sem0).wait()
                    pltpu.sync_copy(rows, acc.at[dblk.at[k]], add=True)
                return carry

            lax.fori_loop(0, NIB, blk, 0)
            plsc.subcore_barrier()

            def wout(i, carry):
                base = s * RT + i * CR
                _build_zidx(zidx, base, CR)
                pltpu.async_copy(acc.at[zidx], srow, sem0).wait()
                pltpu.sync_copy(srow, out_hbm.at[q, s, i])
                return carry

            lax.fori_loop(0, NRC, wout, 0)

    return scat


_scat4 = _make_scatter(4)
_scat2 = _make_scatter(2)


def _dinv_tc(cnt):
    def body(cnt_ref, out_ref):
        deg = 1.0 + cnt_ref[0][:, 0:1] + cnt_ref[1][:, 0:1]
        out_ref[...] = lax.rsqrt(deg)

    return pl.pallas_call(
        body,
        grid=(NBLK,),
        in_specs=[pl.BlockSpec((2, BN, QW), lambda i: (0, i, 0))],
        out_specs=pl.BlockSpec((BN, 1), lambda i: (i, 0)),
        out_shape=jax.ShapeDtypeStruct((NP, 1), jnp.float32),
    )(cnt)


def _embed_tc(xp, embed, W1, dinv):
    def body(x_ref, e_ref, w_ref, d_ref, out_ref):
        t1 = jnp.dot(e_ref[...], w_ref[...], preferred_element_type=jnp.float32)
        iot = lax.broadcasted_iota(jnp.int32, (BN, NUM_TYPES), 1)
        oh = (x_ref[...] == iot).astype(jnp.float32)
        g = jnp.dot(oh, t1, preferred_element_type=jnp.float32) * d_ref[...]
        for q in range(4):
            out_ref[q] = g[:, q * QW:(q + 1) * QW]

    return pl.pallas_call(
        body,
        grid=(NBLK,),
        in_specs=[
            pl.BlockSpec((BN, 1), lambda i: (i, 0)),
            pl.BlockSpec((NUM_TYPES, H), lambda i: (0, 0)),
            pl.BlockSpec((H, H), lambda i: (0, 0)),
            pl.BlockSpec((BN, 1), lambda i: (i, 0)),
        ],
        out_specs=pl.BlockSpec((4, BN, QW), lambda i: (0, i, 0)),
        out_shape=jax.ShapeDtypeStruct((4, NP, QW), jnp.float32),
    )(xp, embed, W1, dinv)


def _layer_tc(S, g, dinv, b, W, nqo):
    nqi = S.shape[0]

    def body(s_ref, g_ref, d_ref, b_ref, w_ref, out_ref):
        d = d_ref[...]
        h = jnp.concatenate(
            [(s_ref[q] + g_ref[q]) * d for q in range(nqi)], axis=1)
        h = jnp.maximum(h + b_ref[...], 0.0)
        go = jnp.dot(h, w_ref[...], preferred_element_type=jnp.float32) * d
        for q in range(nqo):
            out_ref[q] = go[:, q * QW:(q + 1) * QW]

    return pl.pallas_call(
        body,
        grid=(NBLK,),
        in_specs=[
            pl.BlockSpec((nqi, BN, QW), lambda i: (0, i, 0)),
            pl.BlockSpec((nqi, BN, QW), lambda i: (0, i, 0)),
            pl.BlockSpec((BN, 1), lambda i: (i, 0)),
            pl.BlockSpec((1, nqi * QW), lambda i: (0, 0)),
            pl.BlockSpec(W.shape, lambda i: (0, 0)),
        ],
        out_specs=pl.BlockSpec((nqo, BN, QW), lambda i: (0, i, 0)),
        out_shape=jax.ShapeDtypeStruct((nqo, NP, QW), jnp.float32),
    )(S, g, dinv, b, W)


def _out_tc(S, G, dinv, bmu, bls):
    def body(s_ref, g_ref, d_ref, bm_ref, bl_ref, mu_ref, ls_ref):
        d = d_ref[...]
        mu = jnp.concatenate([(s_ref[q] + g_ref[q]) * d for q in (0, 1)], axis=1)
        ls = jnp.concatenate([(s_ref[q] + g_ref[q]) * d for q in (2, 3)], axis=1)
        mu_ref[...] = mu + bm_ref[...]
        ls_ref[...] = ls + bl_ref[...]

    return pl.pallas_call(
        body,
        grid=(NBLK,),
        in_specs=[
            pl.BlockSpec((4, BN, QW), lambda i: (0, i, 0)),
            pl.BlockSpec((4, BN, QW), lambda i: (0, i, 0)),
            pl.BlockSpec((BN, 1), lambda i: (i, 0)),
            pl.BlockSpec((1, OUT), lambda i: (0, 0)),
            pl.BlockSpec((1, OUT), lambda i: (0, 0)),
        ],
        out_specs=[
            pl.BlockSpec((BN, OUT), lambda i: (i, 0)),
            pl.BlockSpec((BN, OUT), lambda i: (i, 0)),
        ],
        out_shape=[
            jax.ShapeDtypeStruct((NP, OUT), jnp.float32),
            jax.ShapeDtypeStruct((NP, OUT), jnp.float32),
        ],
    )(S, G, dinv, bmu, bls)


def kernel(x, edge_index, embed, W1, b1, W2, b2, W3, b3, Wmu, bmu, Wls, bls):
    src = edge_index[0].astype(jnp.int32)
    dst = edge_index[1].astype(jnp.int32)
    padrows = (jnp.arange(EP - E, dtype=jnp.int32) % (NT - N)) + N
    srcp = jnp.concatenate([src, padrows])
    dstp = jnp.concatenate([dst, padrows])
    sidx = srcp.reshape(16, NIB, IB, CE)
    dstS = dstp.reshape(16, NIB, IB, CE)
    dstD = dstp.reshape(16, 2, HCH, 128)
    xp = jnp.concatenate(
        [x.astype(jnp.int32), jnp.zeros((NP - N,), jnp.int32)]).reshape(NP, 1)

    cnt = _deg_sc(dstD)
    dinv = _dinv_tc(cnt)

    g1 = _embed_tc(xp, embed, W1, dinv)
    _g5 = lambda g, nq: g.reshape(nq, 16, NRC, CR, QW)
    _s3 = lambda S, nq: S.reshape(nq, NP, QW)
    S1 = _s3(_scat4(_g5(g1, 4), sidx, dstS), 4)
    g2 = _layer_tc(S1, g1, dinv, b1.reshape(1, H), W2, 4)
    S2 = _s3(_scat4(_g5(g2, 4), sidx, dstS), 4)
    g3 = _layer_tc(S2, g2, dinv, b2.reshape(1, H), W3, 2)
    S3 = _s3(_scat2(_g5(g3, 2), sidx, dstS), 2)
    Wml = jnp.concatenate([Wmu, Wls], axis=1)
    G = _layer_tc(S3, g3, dinv, b3.reshape(1, OUT), Wml, 4)
    S4 = _s3(_scat4(_g5(G, 4), sidx, dstS), 4)
    mu, ls = _out_tc(S4, G, dinv, bmu.reshape(1, OUT), bls.reshape(1, OUT))
    return (mu[:N], ls[:N])
